# baseline (speedup 1.0000x reference)
import jax
import jax.numpy as jnp
from jax import lax
from jax.experimental import pallas as pl
from jax.experimental.pallas import tpu as pltpu
from jax.experimental.pallas import tpu_sc as plsc

_PLACEHOLDER_TOKEN = 500
_L = 16
_CHUNK = 32
_NBUF = 4


def _sc_body(tok_hbm, emb_hbm, vec_hbm, out_hbm,
             bufs, tok_v, vec_v, in_sems, out_sems):
    rows, d = emb_hbm.shape
    nc = 2
    ns = 16
    wid = lax.axis_index("s") * nc + lax.axis_index("c")
    rows_per_tile = rows // (nc * ns)
    base = wid * rows_per_tile
    nchunk = rows_per_tile // _CHUNK
    nvec_per_chunk = _CHUNK // _L

    pltpu.sync_copy(tok_hbm.at[pl.ds(base, rows_per_tile)], tok_v)
    pltpu.sync_copy(vec_hbm.at[0], vec_v)

    def in_dma(t, s):
        return pltpu.make_async_copy(
            emb_hbm.at[pl.ds(base + t * _CHUNK, _CHUNK)], bufs.at[s],
            in_sems.at[s])

    def out_dma(t, s):
        return pltpu.make_async_copy(
            bufs.at[s], out_hbm.at[pl.ds(base + t * _CHUNK, _CHUNK)],
            out_sems.at[s])

    lanes = lax.iota(jnp.int32, _L)

    def fix_rows(s, t):
        for v in range(nvec_per_chunk):
            tok16 = tok_v[pl.ds(t * _CHUNK + v * _L, _L)]
            match = tok16 == _PLACEHOLDER_TOKEN
            m = jnp.where(match, 1, 0)
            any_match = plsc.all_reduce_population_count(match)[0]

            @pl.when(any_match > 0)
            def _():
                def cond(mm):
                    return plsc.all_reduce_population_count(mm > 0)[0] > 0

                def body(mm):
                    lane_v = plsc.all_reduce_ffs(mm > 0)
                    row_v = v * _L + lane_v
                    for k in range(d // _L):
                        plsc.store_scatter(
                            bufs.at[s],
                            [row_v, k * _L + lanes],
                            vec_v[pl.ds(k * _L, _L)])
                    return jnp.where(lanes == lane_v, 0, mm)

                lax.while_loop(cond, body, m)

    def group(g, carry):
        for s in range(_NBUF):
            t = g * _NBUF + s
            in_dma(t, s).wait()
            fix_rows(s, t)
            out_dma(t, s).start()

            sp = (s - 1) % _NBUF

            @pl.when(t >= 1)
            def _():
                out_dma(t - 1, sp).wait()

            @pl.when(t + _NBUF - 1 < nchunk)
            def _():
                in_dma(t + _NBUF - 1, sp).start()

        return carry

    for s in range(_NBUF - 1):
        in_dma(s, s).start()
    lax.fori_loop(0, nchunk // _NBUF, group, 0)
    out_dma(nchunk - 1, (nchunk - 1) % _NBUF).wait()


def kernel(tokenized_text, embedded_text, placeholder_embedding):
    b, n, d = embedded_text.shape
    rows = b * n
    emb = embedded_text.reshape(rows, d)
    tok = tokenized_text.reshape(rows)
    mesh = plsc.VectorSubcoreMesh(core_axis_name="c", subcore_axis_name="s")
    run = pl.kernel(
        _sc_body,
        out_type=jax.ShapeDtypeStruct((rows, d), embedded_text.dtype),
        mesh=mesh,
        scratch_types=[
            pltpu.VMEM((_NBUF, _CHUNK, d), jnp.float32),
            pltpu.VMEM((rows // 32,), jnp.int32),
            pltpu.VMEM((d,), jnp.float32),
            pltpu.SemaphoreType.DMA((_NBUF,)),
            pltpu.SemaphoreType.DMA((_NBUF,)),
        ],
        compiler_params=pltpu.CompilerParams(needs_layout_passes=False),
    )
    out = run(tok, emb, placeholder_embedding)
    return out.reshape(b, n, d)
